# baseline (speedup 1.0000x reference)
import jax
import jax.numpy as jnp
from jax import lax
from jax.experimental import pallas as pl
from jax.experimental.pallas import tpu as pltpu

_LRATE = 0.01
_SIZE = 8192
_CH = 128
_NCH = _SIZE // _CH


def _scaled_mask(x2):
    _SIGN = jnp.int32(-2147483648)
    _LOW31 = jnp.int32(2147483647)
    ib = lax.bitcast_convert_type(x2, jnp.int32)
    key = jnp.where(ib >= 0, ib, ib ^ _LOW31)
    rank = jnp.int32((_SIZE - 1) // 2 + 1)

    io8 = lax.broadcasted_iota(jnp.int32, (8, 1), 0)
    res_b = jnp.int32(0)
    for r in range(8):
        sh = 28 - 4 * r
        d1 = io8 + 1
        d2 = io8 + 9
        t1 = (res_b + (d1 << sh)) ^ _SIGN
        t2 = (res_b + (d2 << sh)) ^ _SIGN
        c1 = jnp.sum((key < t1).astype(jnp.int32), axis=1, keepdims=True)
        c2 = jnp.sum((key < t2).astype(jnp.int32), axis=1, keepdims=True)
        ind1 = (c1 < rank).astype(jnp.int32)
        ind2 = jnp.where(io8 < 7, (c2 < rank).astype(jnp.int32), 0)
        digit = jnp.sum(ind1) + jnp.sum(ind2)
        res_b = res_b + (digit << sh)
    med_s = res_b ^ _SIGN
    med_i = jnp.where(med_s >= 0, med_s, med_s ^ _LOW31)
    med_f = lax.bitcast_convert_type(med_i, jnp.float32)
    return jnp.where(x2 > med_f, jnp.float32(-_LRATE), jnp.float32(0.0))


def _body(x_ref, inp_ref, out_hbm, buf0, buf1, sem):
    y = _scaled_mask(x_ref[...])

    def _chunk(k, buf, slot):
        @pl.when(k >= 2)
        def _():
            pltpu.make_async_copy(
                buf, out_hbm.at[pl.ds((k - 2) * _CH, _CH), :], sem.at[slot]
            ).wait()

        a = inp_ref[:, pl.ds(pl.multiple_of(k * _CH, _CH), _CH)]
        buf[...] = lax.dot_general(
            a, y, (((0,), (0,)), ((), ())),
            preferred_element_type=jnp.float32,
        )
        pltpu.make_async_copy(
            buf, out_hbm.at[pl.ds(k * _CH, _CH), :], sem.at[slot]
        ).start()

    def _step(k, carry):
        @pl.when(k % 2 == 0)
        def _():
            _chunk(k, buf0, 0)

        @pl.when(k % 2 == 1)
        def _():
            _chunk(k, buf1, 1)

        return carry

    lax.fori_loop(0, _NCH, _step, jnp.int32(0))

    pltpu.make_async_copy(
        buf0, out_hbm.at[pl.ds((_NCH - 2) * _CH, _CH), :], sem.at[0]
    ).wait()
    pltpu.make_async_copy(
        buf1, out_hbm.at[pl.ds((_NCH - 1) * _CH, _CH), :], sem.at[1]
    ).wait()


def kernel(x, input):
    x2 = x.reshape(1, _SIZE)
    inp2 = input.reshape(1, _SIZE)
    return pl.pallas_call(
        _body,
        in_specs=[
            pl.BlockSpec(memory_space=pltpu.VMEM),
            pl.BlockSpec(memory_space=pltpu.VMEM),
        ],
        out_specs=pl.BlockSpec(memory_space=pl.ANY),
        out_shape=jax.ShapeDtypeStruct((_SIZE, _SIZE), jnp.float32),
        scratch_shapes=[
            pltpu.VMEM((_CH, _SIZE), jnp.float32),
            pltpu.VMEM((_CH, _SIZE), jnp.float32),
            pltpu.SemaphoreType.DMA((2,)),
        ],
    )(x2, inp2)


# speedup vs baseline: 1.0566x; 1.0566x over previous
import jax
import jax.numpy as jnp
from jax import lax
from jax.experimental import pallas as pl
from jax.experimental.pallas import tpu as pltpu

_LRATE = 0.01
_SIZE = 8192
_ROWS = 256


def _scaled_mask(x2):
    _SIGN = jnp.int32(-2147483648)
    _LOW31 = jnp.int32(2147483647)
    ib = lax.bitcast_convert_type(x2, jnp.int32)
    key = jnp.where(ib >= 0, ib, ib ^ _LOW31)
    rank = jnp.int32((_SIZE - 1) // 2 + 1)

    io8 = lax.broadcasted_iota(jnp.int32, (8, 1), 0)
    res_b = jnp.int32(0)
    for r in range(8):
        sh = 28 - 4 * r
        d1 = io8 + 1
        d2 = io8 + 9
        t1 = (res_b + (d1 << sh)) ^ _SIGN
        t2 = (res_b + (d2 << sh)) ^ _SIGN
        c1 = jnp.sum((key < t1).astype(jnp.int32), axis=1, keepdims=True)
        c2 = jnp.sum((key < t2).astype(jnp.int32), axis=1, keepdims=True)
        ind1 = (c1 < rank).astype(jnp.int32)
        ind2 = jnp.where(io8 < 7, (c2 < rank).astype(jnp.int32), 0)
        digit = jnp.sum(ind1) + jnp.sum(ind2)
        res_b = res_b + (digit << sh)
    med_s = res_b ^ _SIGN
    med_i = jnp.where(med_s >= 0, med_s, med_s ^ _LOW31)
    med_f = lax.bitcast_convert_type(med_i, jnp.float32)
    return jnp.where(x2 > med_f, jnp.float32(-_LRATE), jnp.float32(0.0))


def _body(x_ref, inp_ref, out_ref, y_ref):
    i = pl.program_id(0)

    @pl.when(i == 0)
    def _():
        y_ref[...] = _scaled_mask(x_ref[...])

    a = inp_ref[:, pl.ds(i * _ROWS, _ROWS)]
    out_ref[...] = lax.dot_general(
        a, y_ref[...], (((0,), (0,)), ((), ())),
        preferred_element_type=jnp.float32,
    )


def kernel(x, input):
    x2 = x.reshape(1, _SIZE)
    inp2 = input.reshape(1, _SIZE)
    return pl.pallas_call(
        _body,
        grid=(_SIZE // _ROWS,),
        in_specs=[
            pl.BlockSpec((1, _SIZE), lambda i: (0, 0)),
            pl.BlockSpec((1, _SIZE), lambda i: (0, 0)),
        ],
        out_specs=pl.BlockSpec((_ROWS, _SIZE), lambda i: (i, 0)),
        out_shape=jax.ShapeDtypeStruct((_SIZE, _SIZE), jnp.float32),
        scratch_shapes=[pltpu.VMEM((1, _SIZE), jnp.float32)],
    )(x2, inp2)
